# baseline (speedup 1.0000x reference)
import jax
import jax.numpy as jnp
from jax.experimental import pallas as pl

_B, _N = 4, 512
_IN_NF, _IN_EF, _OUT_F = 16, 8, 16
_TI = 128
_G = 8


def _prep(p_ref):
    f32 = jnp.float32
    bf16 = jnp.bfloat16
    we18 = p_ref[0:8, 0:16]
    we2 = p_ref[8:24, 0:16]
    wn1 = p_ref[24:40, 0:16]
    wn2 = p_ref[40:56, 0:16]
    be1r = p_ref[56:57, 0:16]
    be2r = p_ref[57:58, 0:16]
    bn1r = p_ref[58:59, 0:16]
    bn2r = p_ref[59:60, 0:16]

    tr = lambda lhs, rhs: jax.lax.dot_general(
        lhs, rhs, (((1,), (1,)), ((), ())),
        preferred_element_type=f32)

    r128 = jax.lax.broadcasted_iota(jnp.int32, (128, 16), 0)
    o128 = jax.lax.broadcasted_iota(jnp.int32, (128, 16), 1)
    e1 = jnp.where(r128 % 16 == o128, 1.0, 0.0).astype(f32)
    k8 = jax.lax.broadcasted_iota(jnp.int32, (8, 64), 0)
    c64 = jax.lax.broadcasted_iota(jnp.int32, (8, 64), 1)
    e2 = jnp.where(c64 % 8 == k8, 1.0, 0.0).astype(f32)
    o16 = jax.lax.broadcasted_iota(jnp.int32, (16, 128), 0)
    c128 = jax.lax.broadcasted_iota(jnp.int32, (16, 128), 1)
    e3 = jnp.where(c128 % 16 == o16, 1.0, 0.0).astype(f32)
    ri = jax.lax.broadcasted_iota(jnp.int32, (128, 64), 0)
    ci = jax.lax.broadcasted_iota(jnp.int32, (128, 64), 1)
    m1 = jnp.where(ri // 16 == ci // 8, 1.0, 0.0).astype(f32)
    ri2 = jax.lax.broadcasted_iota(jnp.int32, (128, 128), 0)
    ci2 = jax.lax.broadcasted_iota(jnp.int32, (128, 128), 1)
    m2 = jnp.where(ri2 // 16 == ci2 // 16, 1.0, 0.0).astype(f32)
    ei = jax.lax.broadcasted_iota(jnp.int32, (16, 16), 0)
    ej = jax.lax.broadcasted_iota(jnp.int32, (16, 16), 1)
    eye16 = jnp.where(ei == ej, 1.0, 0.0).astype(f32)

    t1 = tr(e1, we18)
    bd1 = (jnp.dot(t1, e2, preferred_element_type=f32) * m1).astype(bf16)
    t2 = tr(e1, we2)
    bd2 = (jnp.dot(t2, e3, preferred_element_type=f32) * m2).astype(bf16)
    be1 = tr(e1, be1r)
    be2 = tr(e1, be2r)
    wn1t = tr(eye16, wn1)
    wn2t = tr(eye16, wn2)
    bn1 = tr(eye16, bn1r)
    bn2 = tr(eye16, bn2r)
    return bd1, bd2, be1, be2, wn1t, wn2t, bn1, bn2


def _fused_kernel(wt_ref, a_ref, xt_ref, p_ref, wout_ref, xout_ref):
    bf16 = jnp.bfloat16
    bd1, bd2, be1, be2, wn1t, wn2t, bn1, bn2 = _prep(p_ref)

    xt = xt_ref[0]
    h1 = jnp.maximum(
        jnp.dot(wn1t, xt, preferred_element_type=jnp.float32) + bn1, 0.0)
    x1t = jnp.maximum(
        jnp.dot(wn2t, h1, preferred_element_type=jnp.float32) + bn2, 0.0)

    wtb = wt_ref[0].astype(bf16)
    hs = []
    for g in range(_TI // _G):
        rhs = wtb[g * _G:(g + 1) * _G].reshape(_G * _IN_EF, _N)
        h = jnp.maximum(
            jnp.dot(bd1, rhs, preferred_element_type=jnp.float32)
            + be1, 0.0)
        hs.append(h.astype(bf16))
    for g in range(_TI // _G):
        w2 = jnp.maximum(
            jnp.dot(bd2, hs[g], preferred_element_type=jnp.float32)
            + be2, 0.0)
        wout_ref[0, g * _G:(g + 1) * _G] = w2.reshape(_G, _OUT_F, _N)

    a = a_ref[0]
    asum = jnp.sum(a, axis=1, keepdims=True)
    inv = jnp.where(asum == 0.0, 0.0, 1.0 / asum)
    an = a * inv
    wall = wout_ref[0]
    p = wall * x1t[None] * an[:, None, :]
    xnew = jnp.sum(p, axis=2)
    xout_ref[0] = xnew


@jax.jit
def kernel(A, W, x, We1, be1, We2, be2, Wn1, bn1, Wn2, bn2):
    f32 = jnp.float32
    wt = jnp.transpose(W, (0, 1, 3, 2))
    xt = jnp.transpose(x, (0, 2, 1))

    params = jnp.zeros((64, 128), f32)
    params = params.at[0:8, 0:16].set(We1[:_IN_EF])
    params = params.at[8:24, 0:16].set(We2)
    params = params.at[24:40, 0:16].set(Wn1)
    params = params.at[40:56, 0:16].set(Wn2)
    params = params.at[56, 0:16].set(be1)
    params = params.at[57, 0:16].set(be2)
    params = params.at[58, 0:16].set(bn1)
    params = params.at[59, 0:16].set(bn2)

    const = lambda *shape: pl.BlockSpec(shape, lambda b, i: (0,) * len(shape))
    wout, xout = pl.pallas_call(
        _fused_kernel,
        grid=(_B, _N // _TI),
        in_specs=[
            pl.BlockSpec((1, _TI, _IN_EF, _N), lambda b, i: (b, i, 0, 0)),
            pl.BlockSpec((1, _TI, _N), lambda b, i: (b, i, 0)),
            pl.BlockSpec((1, _IN_NF, _N), lambda b, i: (b, 0, 0)),
            const(64, 128),
        ],
        out_specs=[
            pl.BlockSpec((1, _TI, _OUT_F, _N), lambda b, i: (b, i, 0, 0)),
            pl.BlockSpec((1, _TI, _OUT_F), lambda b, i: (b, i, 0)),
        ],
        out_shape=[
            jax.ShapeDtypeStruct((_B, _N, _OUT_F, _N), f32),
            jax.ShapeDtypeStruct((_B, _N, _OUT_F), f32),
        ],
    )(wt, A, xt, params)
    return jnp.transpose(wout, (0, 1, 3, 2)), xout


# speedup vs baseline: 1.0302x; 1.0302x over previous
import jax
import jax.numpy as jnp
from jax.experimental import pallas as pl
from jax.experimental.pallas import tpu as pltpu

_B, _N = 4, 512
_IN_NF, _IN_EF, _OUT_F = 16, 8, 16
_TI = 128
_G = 8


def _prep(p_ref):
    f32 = jnp.float32
    bf16 = jnp.bfloat16
    we18 = p_ref[0:8, 0:16]
    we2 = p_ref[8:24, 0:16]
    wn1 = p_ref[24:40, 0:16]
    wn2 = p_ref[40:56, 0:16]
    be1r = p_ref[56:57, 0:16]
    be2r = p_ref[57:58, 0:16]
    bn1r = p_ref[58:59, 0:16]
    bn2r = p_ref[59:60, 0:16]

    tr = lambda lhs, rhs: jax.lax.dot_general(
        lhs, rhs, (((1,), (1,)), ((), ())),
        preferred_element_type=f32)

    r128 = jax.lax.broadcasted_iota(jnp.int32, (128, 16), 0)
    o128 = jax.lax.broadcasted_iota(jnp.int32, (128, 16), 1)
    e1 = jnp.where(r128 % 16 == o128, 1.0, 0.0).astype(f32)
    k8 = jax.lax.broadcasted_iota(jnp.int32, (8, 64), 0)
    c64 = jax.lax.broadcasted_iota(jnp.int32, (8, 64), 1)
    e2 = jnp.where(c64 % 8 == k8, 1.0, 0.0).astype(f32)
    o16 = jax.lax.broadcasted_iota(jnp.int32, (16, 128), 0)
    c128 = jax.lax.broadcasted_iota(jnp.int32, (16, 128), 1)
    e3 = jnp.where(c128 % 16 == o16, 1.0, 0.0).astype(f32)
    ri = jax.lax.broadcasted_iota(jnp.int32, (128, 64), 0)
    ci = jax.lax.broadcasted_iota(jnp.int32, (128, 64), 1)
    m1 = jnp.where(ri // 16 == ci // 8, 1.0, 0.0).astype(f32)
    ri2 = jax.lax.broadcasted_iota(jnp.int32, (128, 128), 0)
    ci2 = jax.lax.broadcasted_iota(jnp.int32, (128, 128), 1)
    m2 = jnp.where(ri2 // 16 == ci2 // 16, 1.0, 0.0).astype(f32)
    ei = jax.lax.broadcasted_iota(jnp.int32, (16, 16), 0)
    ej = jax.lax.broadcasted_iota(jnp.int32, (16, 16), 1)
    eye16 = jnp.where(ei == ej, 1.0, 0.0).astype(f32)

    t1 = tr(e1, we18)
    bd1 = (jnp.dot(t1, e2, preferred_element_type=f32) * m1).astype(bf16)
    t2 = tr(e1, we2)
    bd2 = (jnp.dot(t2, e3, preferred_element_type=f32) * m2).astype(bf16)
    be1 = tr(e1, be1r)
    be2 = tr(e1, be2r)
    wn1t = tr(eye16, wn1)
    wn2t = tr(eye16, wn2)
    bn1 = tr(eye16, bn1r)
    bn2 = tr(eye16, bn2r)
    return bd1, bd2, be1, be2, wn1t, wn2t, bn1, bn2


def _fused_kernel(wt_ref, a_ref, xt_ref, p_ref, wout_ref, xout_ref,
                  s_bd1, s_bd2, s_be, s_nw, s_nb):
    bf16 = jnp.bfloat16

    @pl.when(jnp.logical_and(pl.program_id(0) == 0, pl.program_id(1) == 0))
    def _do_prep():
        bd1_, bd2_, be1_, be2_, wn1t_, wn2t_, bn1_, bn2_ = _prep(p_ref)
        s_bd1[...] = bd1_
        s_bd2[...] = bd2_
        s_be[:, 0:1] = be1_
        s_be[:, 1:2] = be2_
        s_nw[0:16] = wn1t_
        s_nw[16:32] = wn2t_
        s_nb[0:16] = bn1_
        s_nb[16:32] = bn2_

    bd1 = s_bd1[...]
    bd2 = s_bd2[...]
    be1 = s_be[:, 0:1]
    be2 = s_be[:, 1:2]
    wn1t = s_nw[0:16]
    wn2t = s_nw[16:32]
    bn1 = s_nb[0:16]
    bn2 = s_nb[16:32]

    xt = xt_ref[0]
    h1 = jnp.maximum(
        jnp.dot(wn1t, xt, preferred_element_type=jnp.float32) + bn1, 0.0)
    x1t = jnp.maximum(
        jnp.dot(wn2t, h1, preferred_element_type=jnp.float32) + bn2, 0.0)

    wtb = wt_ref[0].astype(bf16)
    hs = []
    for g in range(_TI // _G):
        rhs = wtb[g * _G:(g + 1) * _G].reshape(_G * _IN_EF, _N)
        h = jnp.maximum(
            jnp.dot(bd1, rhs, preferred_element_type=jnp.float32)
            + be1, 0.0)
        hs.append(h.astype(bf16))
    for g in range(_TI // _G):
        w2 = jnp.maximum(
            jnp.dot(bd2, hs[g], preferred_element_type=jnp.float32)
            + be2, 0.0)
        wout_ref[0, g * _G:(g + 1) * _G] = w2.reshape(_G, _OUT_F, _N)

    a = a_ref[0]
    asum = jnp.sum(a, axis=1, keepdims=True)
    inv = jnp.where(asum == 0.0, 0.0, 1.0 / asum)
    an = a * inv
    wall = wout_ref[0]
    p = wall * x1t[None] * an[:, None, :]
    xnew = jnp.sum(p, axis=2)
    xout_ref[0] = xnew


@jax.jit
def kernel(A, W, x, We1, be1, We2, be2, Wn1, bn1, Wn2, bn2):
    f32 = jnp.float32
    wt = jnp.transpose(W, (0, 1, 3, 2))
    xt = jnp.transpose(x, (0, 2, 1))

    params = jnp.zeros((64, 128), f32)
    params = params.at[0:8, 0:16].set(We1[:_IN_EF])
    params = params.at[8:24, 0:16].set(We2)
    params = params.at[24:40, 0:16].set(Wn1)
    params = params.at[40:56, 0:16].set(Wn2)
    params = params.at[56, 0:16].set(be1)
    params = params.at[57, 0:16].set(be2)
    params = params.at[58, 0:16].set(bn1)
    params = params.at[59, 0:16].set(bn2)

    const = lambda *shape: pl.BlockSpec(shape, lambda b, i: (0,) * len(shape))
    wout, xout = pl.pallas_call(
        _fused_kernel,
        grid=(_B, _N // _TI),
        in_specs=[
            pl.BlockSpec((1, _TI, _IN_EF, _N), lambda b, i: (b, i, 0, 0)),
            pl.BlockSpec((1, _TI, _N), lambda b, i: (b, i, 0)),
            pl.BlockSpec((1, _IN_NF, _N), lambda b, i: (b, 0, 0)),
            const(64, 128),
        ],
        out_specs=[
            pl.BlockSpec((1, _TI, _OUT_F, _N), lambda b, i: (b, i, 0, 0)),
            pl.BlockSpec((1, _TI, _OUT_F), lambda b, i: (b, i, 0)),
        ],
        out_shape=[
            jax.ShapeDtypeStruct((_B, _N, _OUT_F, _N), f32),
            jax.ShapeDtypeStruct((_B, _N, _OUT_F), f32),
        ],
        scratch_shapes=[
            pltpu.VMEM((128, 64), jnp.bfloat16),
            pltpu.VMEM((128, 128), jnp.bfloat16),
            pltpu.VMEM((128, 2), f32),
            pltpu.VMEM((32, 16), f32),
            pltpu.VMEM((32, 1), f32),
        ],
    )(wt, A, xt, params)
    return jnp.transpose(wout, (0, 1, 3, 2)), xout
